# baseline (speedup 1.0000x reference)
import functools

import jax
import jax.numpy as jnp
import numpy as np
from jax import lax
from jax.experimental import pallas as pl
from jax.experimental.pallas import tpu as pltpu
from jax.experimental.pallas import tpu_sc as plsc

_FIELD = 26
_FEAT = 10000
_K = 8
_D = _FIELD * _K
_B = 4096
_NROW = _FIELD * _FEAT
_NC = 2
_NS = 16
_NW = _NC * _NS
_BPW = _B // _NW
_NVREG = _D // 16

_FB = _FIELD // 2
_NBE = 8
_CIDX = _NBE * _FB
_NCHUNK = _BPW // _NBE
_WPAD = 32

_TBN = 8192
_NBLK0 = 16
_SPLIT = _NBLK0 * _TBN - _TBN
_N0 = _NBLK0 * _TBN
_N1 = _NROW - _SPLIT
_NBLK1 = pl.cdiv(_N1, _TBN)

_OFFS = np.arange(_FIELD, dtype=np.int32) * _FEAT

_MESH = plsc.VectorSubcoreMesh(
    core_axis_name="c", subcore_axis_name="s", num_cores=_NC, num_subcores=_NS
)
_SC_PARAMS = pltpu.CompilerParams(
    needs_layout_passes=False, use_tc_tiling_on_sc=False
)


@functools.partial(
    pl.kernel,
    out_type=jax.ShapeDtypeStruct((_B, _D), jnp.float32),
    mesh=_MESH,
    compiler_params=_SC_PARAMS,
    scratch_types=[
        pltpu.VMEM((_NCHUNK, _CIDX), jnp.int32),
        pltpu.VMEM((_CIDX, _D), jnp.float32),
        pltpu.VMEM((_CIDX, _D), jnp.float32),
        pltpu.VMEM((_BPW, _D), jnp.float32),
        pltpu.SemaphoreType.DMA,
        pltpu.SemaphoreType.DMA,
    ],
)
def _ffm_band0(idx_hbm, vtab_hbm, lat_hbm, idx_l, vb0, vb1, lat_l, sv0, sv1):
    wid = lax.axis_index("s") * _NC + lax.axis_index("c")
    base = wid * _BPW
    pltpu.sync_copy(idx_hbm.at[pl.ds(wid * _NCHUNK, _NCHUNK)], idx_l)

    vbufs = (vb0, vb1)
    svs = (sv0, sv1)

    def fire(c, b):
        pltpu.async_copy(vtab_hbm.at[idx_l.at[c]], vbufs[b], svs[b])

    fire(0, 0)
    fire(1, 1)

    def do_chunk(c, b):
        vb = vbufs[b]
        pltpu.make_async_copy(vtab_hbm.at[idx_l.at[c]], vb, svs[b]).wait()
        for e in range(_NBE):
            r0 = e * _FB
            accs = tuple(vb[r0, pl.ds(16 * t, 16)] for t in range(_NVREG))

            def red(i, accs):
                r = r0 + 1 + i * 3
                for k in range(3):
                    accs = tuple(
                        accs[t] + vb[r + k, pl.ds(16 * t, 16)]
                        for t in range(_NVREG)
                    )
                return accs

            accs = lax.fori_loop(0, 4, red, accs)
            le = c * _NBE + e
            for t in range(_NVREG):
                lat_l[le, pl.ds(16 * t, 16)] = accs[t]

        @pl.when(c + 2 < _NCHUNK)
        def _():
            fire(c + 2, b)

    def it(i, carry):
        do_chunk(2 * i, 0)
        do_chunk(2 * i + 1, 1)
        return carry

    lax.fori_loop(0, _NCHUNK // 2, it, 0)
    pltpu.sync_copy(lat_l, lat_hbm.at[pl.ds(base, _BPW)])


@functools.partial(
    pl.kernel,
    out_type=jax.ShapeDtypeStruct((_B,), jnp.float32),
    mesh=_MESH,
    compiler_params=_SC_PARAMS,
    scratch_types=[
        pltpu.VMEM((_NCHUNK, _CIDX), jnp.int32),
        pltpu.VMEM((2 * _NCHUNK, 128), jnp.int32),
        pltpu.VMEM((_CIDX, _D), jnp.float32),
        pltpu.VMEM((_CIDX, _D), jnp.float32),
        pltpu.VMEM((_NBE * _WPAD,), jnp.float32),
        pltpu.VMEM((_NBE * _WPAD,), jnp.float32),
        pltpu.VMEM((_BPW, _D), jnp.float32),
        pltpu.VMEM((_BPW,), jnp.float32),
        pltpu.VMEM((24,), jnp.float32),
        pltpu.SemaphoreType.DMA,
        pltpu.SemaphoreType.DMA,
        pltpu.SemaphoreType.DMA,
        pltpu.SemaphoreType.DMA,
    ],
)
def _ffm_band1(idx_hbm, idxw_hbm, vtab_hbm, wtab_hbm, lat_hbm, out_hbm,
               idx_l, idxw_l, vb0, vb1, wb0, wb1, lat_l, out_l, fold,
               sv0, sv1, sw0, sw1):
    wid = lax.axis_index("s") * _NC + lax.axis_index("c")
    base = wid * _BPW
    pltpu.sync_copy(idx_hbm.at[pl.ds(wid * _NCHUNK, _NCHUNK)], idx_l)
    pltpu.sync_copy(idxw_hbm.at[pl.ds(wid * 2 * _NCHUNK, 2 * _NCHUNK)], idxw_l)
    pltpu.sync_copy(lat_hbm.at[pl.ds(base, _BPW)], lat_l)
    fold[pl.ds(8, 16)] = jnp.zeros((16,), jnp.float32)

    vbufs = (vb0, vb1)
    wbufs = (wb0, wb1)
    svs = (sv0, sv1)
    sws = (sw0, sw1)
    lane_ids = lax.iota(jnp.int32, 16)
    lane_lo = lane_ids < 8
    lane_w = lane_ids < (_FIELD - 16)

    def fire(c, b):
        pltpu.async_copy(vtab_hbm.at[idx_l.at[c]], vbufs[b], svs[b])
        pltpu.async_copy(
            wtab_hbm.at[idxw_l.at[2 * c]], wbufs[b].at[pl.ds(0, 128)], sws[b]
        )
        pltpu.async_copy(
            wtab_hbm.at[idxw_l.at[2 * c + 1]], wbufs[b].at[pl.ds(128, 128)],
            sws[b],
        )

    fire(0, 0)
    fire(1, 1)

    def do_chunk(c, b):
        vb = vbufs[b]
        wb = wbufs[b]
        pltpu.make_async_copy(vtab_hbm.at[idx_l.at[c]], vb, svs[b]).wait()
        pltpu.make_async_copy(
            wtab_hbm.at[idxw_l.at[2 * c]], wb.at[pl.ds(0, 128)], sws[b]
        ).wait()
        pltpu.make_async_copy(
            wtab_hbm.at[idxw_l.at[2 * c + 1]], wb.at[pl.ds(128, 128)], sws[b]
        ).wait()
        for e in range(_NBE):
            r0 = e * _FB
            le = c * _NBE + e
            accs = tuple(
                lat_l[le, pl.ds(16 * t, 16)] + vb[r0, pl.ds(16 * t, 16)]
                for t in range(_NVREG)
            )

            def red(i, accs):
                r = r0 + 1 + i * 3
                for k in range(3):
                    accs = tuple(
                        accs[t] + vb[r + k, pl.ds(16 * t, 16)]
                        for t in range(_NVREG)
                    )
                return accs

            accs = lax.fori_loop(0, 4, red, accs)

            g = accs[0]
            q = accs[0] * accs[0]
            for t in range(1, _NVREG):
                g = g + accs[t]
                q = q + accs[t] * accs[t]
            sumsq = jnp.sum(q)
            fold[pl.ds(0, 16)] = g
            h = fold[pl.ds(8, 16)]
            s_ext = g + h
            s_m = jnp.where(lane_lo, s_ext, 0.0)
            s2 = jnp.sum(s_m * s_m)
            second = 0.5 * (s2 - sumsq)

            wa = wb[pl.ds(e * _WPAD, 16)]
            wbv = wb[pl.ds(e * _WPAD + 16, 16)]
            ws = jnp.sum(wa + jnp.where(lane_w, wbv, 0.0))
            slot = (c // 2) * 16
            pos = (c % 2) * _NBE + e
            cur = out_l[pl.ds(slot, 16)]
            out_l[pl.ds(slot, 16)] = jnp.where(
                lane_ids == pos, ws + second, cur
            )

        @pl.when(c + 2 < _NCHUNK)
        def _():
            fire(c + 2, b)

    def it(i, carry):
        do_chunk(2 * i, 0)
        do_chunk(2 * i + 1, 1)
        return carry

    lax.fori_loop(0, _NCHUNK // 2, it, 0)
    pltpu.sync_copy(out_l, out_hbm.at[pl.ds(base, _BPW)])


def _tr_body(x_ref, o_ref):
    o_ref[...] = x_ref[...].T


_transpose0 = pl.pallas_call(
    _tr_body,
    grid=(_NBLK0,),
    in_specs=[pl.BlockSpec((_D, _TBN), lambda j: (0, j))],
    out_specs=pl.BlockSpec((_TBN, _D), lambda j: (j, 0)),
    out_shape=jax.ShapeDtypeStruct((_N0, _D), jnp.float32),
)

_transpose1 = pl.pallas_call(
    _tr_body,
    grid=(_NBLK1,),
    in_specs=[pl.BlockSpec((_D, _TBN), lambda j: (0, j + _NBLK0 - 1))],
    out_specs=pl.BlockSpec((_TBN, _D), lambda j: (j, 0)),
    out_shape=jax.ShapeDtypeStruct((_N1, _D), jnp.float32),
)


@jax.jit
def kernel(inputs, w0, w, v):
    mapped = jnp.asarray(inputs, jnp.int32) + jnp.asarray(_OFFS)[None, :]
    idx0 = mapped[:, :_FB].reshape(_B * _FB // _CIDX, _CIDX)
    idx1 = (mapped[:, _FB:] - _SPLIT).reshape(_B * _FB // _CIDX, _CIDX)
    idxw = jnp.concatenate(
        [mapped, jnp.zeros((_B, _WPAD - _FIELD), jnp.int32)], axis=1
    ).reshape(_B * _WPAD // 128, 128)
    vt = v.reshape(_NROW, _D).T
    x0 = _transpose0(vt)
    x1 = _transpose1(vt)
    lat = _ffm_band0(idx0, x0)
    out = _ffm_band1(idx1, idxw, x1, w.reshape(_NROW), lat)
    return out.reshape(_B, 1) + w0


# speedup vs baseline: 1.0589x; 1.0589x over previous
import functools

import jax
import jax.numpy as jnp
import numpy as np
from jax import lax
from jax.experimental import pallas as pl
from jax.experimental.pallas import tpu as pltpu
from jax.experimental.pallas import tpu_sc as plsc

_FIELD = 26
_FEAT = 10000
_K = 8
_D = _FIELD * _K
_B = 4096
_NROW = _FIELD * _FEAT
_NC = 2
_NS = 16
_NW = _NC * _NS
_BPW = _B // _NW
_NVREG = _D // 16

_FB = _FIELD // 2
_NBE = 8
_CIDX = _NBE * _FB
_NCHUNK = _BPW // _NBE
_WPAD = 32

_NPAIR = (_NVREG + 1) // 2
_DU = _NPAIR * 16

_TBN = 8192
_NBLK0 = 16
_SPLIT = _NBLK0 * _TBN - _TBN
_N0 = _NBLK0 * _TBN
_N1 = _NROW - _SPLIT
_NBLK1 = pl.cdiv(_N1, _TBN)

_OFFS = np.arange(_FIELD, dtype=np.int32) * _FEAT

_MESH = plsc.VectorSubcoreMesh(
    core_axis_name="c", subcore_axis_name="s", num_cores=_NC, num_subcores=_NS
)
_SC_PARAMS = pltpu.CompilerParams(
    needs_layout_passes=False, use_tc_tiling_on_sc=False
)


def _row_add(vb, row, accs):
    new = list(accs)
    for u in range(_NPAIR):
        wv = vb[row, pl.ds(16 * u, 16)]
        ab = plsc.bitcast(wv, jnp.bfloat16)
        a, b = plsc.unpack(ab, format=plsc.PackFormat.INTERLEAVED)
        new[2 * u] = new[2 * u] + a
        if 2 * u + 1 < _NVREG:
            new[2 * u + 1] = new[2 * u + 1] + b
    return tuple(new)


@functools.partial(
    pl.kernel,
    out_type=jax.ShapeDtypeStruct((_B, _D), jnp.float32),
    mesh=_MESH,
    compiler_params=_SC_PARAMS,
    scratch_types=[
        pltpu.VMEM((_NCHUNK, _CIDX), jnp.int32),
        pltpu.VMEM((_CIDX, _DU), jnp.uint32),
        pltpu.VMEM((_CIDX, _DU), jnp.uint32),
        pltpu.VMEM((_BPW, _D), jnp.float32),
        pltpu.SemaphoreType.DMA,
        pltpu.SemaphoreType.DMA,
    ],
)
def _ffm_band0(idx_hbm, vtab_hbm, lat_hbm, idx_l, vb0, vb1, lat_l, sv0, sv1):
    wid = lax.axis_index("s") * _NC + lax.axis_index("c")
    base = wid * _BPW
    pltpu.sync_copy(idx_hbm.at[pl.ds(wid * _NCHUNK, _NCHUNK)], idx_l)

    vbufs = (vb0, vb1)
    svs = (sv0, sv1)

    def fire(c, b):
        pltpu.async_copy(vtab_hbm.at[idx_l.at[c]], vbufs[b], svs[b])

    fire(0, 0)
    fire(1, 1)

    def do_chunk(c, b):
        vb = vbufs[b]
        pltpu.make_async_copy(vtab_hbm.at[idx_l.at[c]], vb, svs[b]).wait()
        zero = jnp.zeros((16,), jnp.float32)
        for e in range(_NBE):
            r0 = e * _FB
            accs = lax.fori_loop(
                0, _FB,
                lambda i, accs: _row_add(vb, r0 + i, accs),
                (zero,) * _NVREG,
            )
            le = c * _NBE + e
            for t in range(_NVREG):
                lat_l[le, pl.ds(16 * t, 16)] = accs[t]

        @pl.when(c + 2 < _NCHUNK)
        def _():
            fire(c + 2, b)

    def it(i, carry):
        do_chunk(2 * i, 0)
        do_chunk(2 * i + 1, 1)
        return carry

    lax.fori_loop(0, _NCHUNK // 2, it, 0)
    pltpu.sync_copy(lat_l, lat_hbm.at[pl.ds(base, _BPW)])


@functools.partial(
    pl.kernel,
    out_type=jax.ShapeDtypeStruct((_B,), jnp.float32),
    mesh=_MESH,
    compiler_params=_SC_PARAMS,
    scratch_types=[
        pltpu.VMEM((_NCHUNK, _CIDX), jnp.int32),
        pltpu.VMEM((2 * _NCHUNK, 128), jnp.int32),
        pltpu.VMEM((_CIDX, _DU), jnp.uint32),
        pltpu.VMEM((_CIDX, _DU), jnp.uint32),
        pltpu.VMEM((_NBE * _WPAD,), jnp.float32),
        pltpu.VMEM((_NBE * _WPAD,), jnp.float32),
        pltpu.VMEM((_BPW, _D), jnp.float32),
        pltpu.VMEM((_BPW,), jnp.float32),
        pltpu.VMEM((24,), jnp.float32),
        pltpu.SemaphoreType.DMA,
        pltpu.SemaphoreType.DMA,
        pltpu.SemaphoreType.DMA,
        pltpu.SemaphoreType.DMA,
    ],
)
def _ffm_band1(idx_hbm, idxw_hbm, vtab_hbm, wtab_hbm, lat_hbm, out_hbm,
               idx_l, idxw_l, vb0, vb1, wb0, wb1, lat_l, out_l, fold,
               sv0, sv1, sw0, sw1):
    wid = lax.axis_index("s") * _NC + lax.axis_index("c")
    base = wid * _BPW
    pltpu.sync_copy(idx_hbm.at[pl.ds(wid * _NCHUNK, _NCHUNK)], idx_l)
    pltpu.sync_copy(idxw_hbm.at[pl.ds(wid * 2 * _NCHUNK, 2 * _NCHUNK)], idxw_l)
    pltpu.sync_copy(lat_hbm.at[pl.ds(base, _BPW)], lat_l)
    fold[pl.ds(8, 16)] = jnp.zeros((16,), jnp.float32)

    vbufs = (vb0, vb1)
    wbufs = (wb0, wb1)
    svs = (sv0, sv1)
    sws = (sw0, sw1)
    lane_ids = lax.iota(jnp.int32, 16)
    lane_lo = lane_ids < 8
    lane_w = lane_ids < (_FIELD - 16)

    def fire(c, b):
        pltpu.async_copy(vtab_hbm.at[idx_l.at[c]], vbufs[b], svs[b])
        pltpu.async_copy(
            wtab_hbm.at[idxw_l.at[2 * c]], wbufs[b].at[pl.ds(0, 128)], sws[b]
        )
        pltpu.async_copy(
            wtab_hbm.at[idxw_l.at[2 * c + 1]], wbufs[b].at[pl.ds(128, 128)],
            sws[b],
        )

    fire(0, 0)
    fire(1, 1)

    def do_chunk(c, b):
        vb = vbufs[b]
        wb = wbufs[b]
        pltpu.make_async_copy(vtab_hbm.at[idx_l.at[c]], vb, svs[b]).wait()
        pltpu.make_async_copy(
            wtab_hbm.at[idxw_l.at[2 * c]], wb.at[pl.ds(0, 128)], sws[b]
        ).wait()
        pltpu.make_async_copy(
            wtab_hbm.at[idxw_l.at[2 * c + 1]], wb.at[pl.ds(128, 128)], sws[b]
        ).wait()
        for e in range(_NBE):
            r0 = e * _FB
            le = c * _NBE + e
            accs = tuple(
                lat_l[le, pl.ds(16 * t, 16)] for t in range(_NVREG)
            )
            accs = lax.fori_loop(
                0, _FB, lambda i, accs: _row_add(vb, r0 + i, accs), accs
            )

            g = accs[0]
            q = accs[0] * accs[0]
            for t in range(1, _NVREG):
                g = g + accs[t]
                q = q + accs[t] * accs[t]
            sumsq = jnp.sum(q)
            fold[pl.ds(0, 16)] = g
            h = fold[pl.ds(8, 16)]
            s_ext = g + h
            s_m = jnp.where(lane_lo, s_ext, 0.0)
            s2 = jnp.sum(s_m * s_m)
            second = 0.5 * (s2 - sumsq)

            wa = wb[pl.ds(e * _WPAD, 16)]
            wbv = wb[pl.ds(e * _WPAD + 16, 16)]
            ws = jnp.sum(wa + jnp.where(lane_w, wbv, 0.0))
            slot = (c // 2) * 16
            pos = (c % 2) * _NBE + e
            cur = out_l[pl.ds(slot, 16)]
            out_l[pl.ds(slot, 16)] = jnp.where(
                lane_ids == pos, ws + second, cur
            )

        @pl.when(c + 2 < _NCHUNK)
        def _():
            fire(c + 2, b)

    def it(i, carry):
        do_chunk(2 * i, 0)
        do_chunk(2 * i + 1, 1)
        return carry

    lax.fori_loop(0, _NCHUNK // 2, it, 0)
    pltpu.sync_copy(out_l, out_hbm.at[pl.ds(base, _BPW)])


def _tr_body(x_ref, o_ref):
    xt = x_ref[...].T.astype(jnp.bfloat16)
    parts = []
    for c in range(_NPAIR):
        a = xt[:, 32 * c:32 * c + 16]
        au = lax.bitcast_convert_type(a, jnp.uint16).astype(jnp.uint32)
        if 2 * c + 1 < _NVREG:
            b = xt[:, 32 * c + 16:32 * c + 32]
            bu = lax.bitcast_convert_type(b, jnp.uint16).astype(jnp.uint32)
            au = au | (bu << 16)
        parts.append(au)
    o_ref[...] = jnp.concatenate(parts, axis=1)


_transpose0 = pl.pallas_call(
    _tr_body,
    grid=(_NBLK0,),
    in_specs=[pl.BlockSpec((_D, _TBN), lambda j: (0, j))],
    out_specs=pl.BlockSpec((_TBN, _DU), lambda j: (j, 0)),
    out_shape=jax.ShapeDtypeStruct((_N0, _DU), jnp.uint32),
)

_transpose1 = pl.pallas_call(
    _tr_body,
    grid=(_NBLK1,),
    in_specs=[pl.BlockSpec((_D, _TBN), lambda j: (0, j + _NBLK0 - 1))],
    out_specs=pl.BlockSpec((_TBN, _DU), lambda j: (j, 0)),
    out_shape=jax.ShapeDtypeStruct((_N1, _DU), jnp.uint32),
)


@jax.jit
def kernel(inputs, w0, w, v):
    mapped = jnp.asarray(inputs, jnp.int32) + jnp.asarray(_OFFS)[None, :]
    idx0 = mapped[:, :_FB].reshape(_B * _FB // _CIDX, _CIDX)
    idx1 = (mapped[:, _FB:] - _SPLIT).reshape(_B * _FB // _CIDX, _CIDX)
    idxw = jnp.concatenate(
        [mapped, jnp.zeros((_B, _WPAD - _FIELD), jnp.int32)], axis=1
    ).reshape(_B * _WPAD // 128, 128)
    vt = v.reshape(_NROW, _D).T
    x0 = _transpose0(vt)
    x1 = _transpose1(vt)
    lat = _ffm_band0(idx0, x0)
    out = _ffm_band1(idx1, idxw, x1, w.reshape(_NROW), lat)
    return out.reshape(_B, 1) + w0


# speedup vs baseline: 1.4897x; 1.4069x over previous
import functools

import jax
import jax.numpy as jnp
import numpy as np
from jax import lax
from jax.experimental import pallas as pl
from jax.experimental.pallas import tpu as pltpu
from jax.experimental.pallas import tpu_sc as plsc

_FIELD = 26
_FEAT = 10000
_K = 8
_D = _FIELD * _K
_B = 4096
_NROW = _FIELD * _FEAT
_NC = 2
_NS = 16
_NW = _NC * _NS
_BPW = _B // _NW
_NVREG = _D // 16

_FB = _FIELD // 2
_NBE = 8
_CIDX = _NBE * _FB
_NCHUNK = _BPW // _NBE
_WPAD = 32

_NPAIR = (_NVREG + 1) // 2
_DU = _NPAIR * 16

_TBN = 8192
_NBLK0 = 16
_SPLIT = _NBLK0 * _TBN - _TBN
_N0 = _NBLK0 * _TBN
_N1 = _NROW - _SPLIT
_NBLK1 = pl.cdiv(_N1, _TBN)

_OFFS = np.arange(_FIELD, dtype=np.int32) * _FEAT

_MESH = plsc.VectorSubcoreMesh(
    core_axis_name="c", subcore_axis_name="s", num_cores=_NC, num_subcores=_NS
)
_SC_PARAMS = pltpu.CompilerParams(
    needs_layout_passes=False, use_tc_tiling_on_sc=False
)


def _row_add(vb, row, accs):
    new = list(accs)
    for u in range(_NPAIR):
        wv = vb[row, pl.ds(16 * u, 16)]
        ab = plsc.bitcast(wv, jnp.bfloat16)
        a, b = plsc.unpack(ab, format=plsc.PackFormat.INTERLEAVED)
        new[2 * u] = new[2 * u] + a
        if 2 * u + 1 < _NVREG:
            new[2 * u + 1] = new[2 * u + 1] + b
    return tuple(new)


@functools.partial(
    pl.kernel,
    out_type=jax.ShapeDtypeStruct((_B, _D), jnp.float32),
    mesh=_MESH,
    compiler_params=_SC_PARAMS,
    scratch_types=[
        pltpu.VMEM((_NCHUNK, _CIDX), jnp.int32),
        pltpu.VMEM((_CIDX, _DU), jnp.uint32),
        pltpu.VMEM((_CIDX, _DU), jnp.uint32),
        pltpu.VMEM((_BPW, _D), jnp.float32),
        pltpu.SemaphoreType.DMA,
        pltpu.SemaphoreType.DMA,
    ],
)
def _ffm_band0(idx_hbm, vtab_hbm, lat_hbm, idx_l, vb0, vb1, lat_l, sv0, sv1):
    wid = lax.axis_index("s") * _NC + lax.axis_index("c")
    base = wid * _BPW
    pltpu.sync_copy(idx_hbm.at[pl.ds(wid * _NCHUNK, _NCHUNK)], idx_l)

    vbufs = (vb0, vb1)
    svs = (sv0, sv1)

    def fire(c, b):
        pltpu.async_copy(vtab_hbm.at[idx_l.at[c]], vbufs[b], svs[b])

    fire(0, 0)
    fire(1, 1)

    def do_chunk(c, b):
        vb = vbufs[b]
        pltpu.make_async_copy(vtab_hbm.at[idx_l.at[c]], vb, svs[b]).wait()
        zero = jnp.zeros((16,), jnp.float32)
        for e in range(_NBE):
            r0 = e * _FB
            accs = lax.fori_loop(
                0, _FB,
                lambda i, accs: _row_add(vb, r0 + i, accs),
                (zero,) * _NVREG,
            )
            le = c * _NBE + e
            for t in range(_NVREG):
                lat_l[le, pl.ds(16 * t, 16)] = accs[t]

        @pl.when(c + 2 < _NCHUNK)
        def _():
            fire(c + 2, b)

    def it(i, carry):
        do_chunk(2 * i, 0)
        do_chunk(2 * i + 1, 1)
        return carry

    lax.fori_loop(0, _NCHUNK // 2, it, 0)
    pltpu.sync_copy(lat_l, lat_hbm.at[pl.ds(base, _BPW)])


@functools.partial(
    pl.kernel,
    out_type=jax.ShapeDtypeStruct((_B,), jnp.float32),
    mesh=_MESH,
    compiler_params=_SC_PARAMS,
    scratch_types=[
        pltpu.VMEM((_NCHUNK, _CIDX), jnp.int32),
        pltpu.VMEM((2 * _NCHUNK, 128), jnp.int32),
        pltpu.VMEM((_CIDX, _DU), jnp.uint32),
        pltpu.VMEM((_CIDX, _DU), jnp.uint32),
        pltpu.VMEM((_NBE * _WPAD,), jnp.float32),
        pltpu.VMEM((_NBE * _WPAD,), jnp.float32),
        pltpu.VMEM((_BPW, _D), jnp.float32),
        pltpu.VMEM((_BPW,), jnp.float32),
        pltpu.VMEM((256,), jnp.float32),
        pltpu.VMEM((256,), jnp.float32),
        pltpu.SemaphoreType.DMA,
        pltpu.SemaphoreType.DMA,
        pltpu.SemaphoreType.DMA,
        pltpu.SemaphoreType.DMA,
    ],
)
def _ffm_band1(idx_hbm, idxw_hbm, vtab_hbm, wtab_hbm, lat_hbm, out_hbm,
               idx_l, idxw_l, vb0, vb1, wb0, wb1, lat_l, out_l, tb, gb,
               sv0, sv1, sw0, sw1):
    wid = lax.axis_index("s") * _NC + lax.axis_index("c")
    base = wid * _BPW
    pltpu.sync_copy(idx_hbm.at[pl.ds(wid * _NCHUNK, _NCHUNK)], idx_l)
    pltpu.sync_copy(idxw_hbm.at[pl.ds(wid * 2 * _NCHUNK, 2 * _NCHUNK)], idxw_l)
    pltpu.sync_copy(lat_hbm.at[pl.ds(base, _BPW)], lat_l)

    vbufs = (vb0, vb1)
    wbufs = (wb0, wb1)
    svs = (sv0, sv1)
    sws = (sw0, sw1)
    lane_ids = lax.iota(jnp.int32, 16)
    lane_w = lane_ids < (_FIELD - 16)
    col_idx = lane_ids * 16

    def fire(c, b):
        pltpu.async_copy(vtab_hbm.at[idx_l.at[c]], vbufs[b], svs[b])
        pltpu.async_copy(
            wtab_hbm.at[idxw_l.at[2 * c]], wbufs[b].at[pl.ds(0, 128)], sws[b]
        )
        pltpu.async_copy(
            wtab_hbm.at[idxw_l.at[2 * c + 1]], wbufs[b].at[pl.ds(128, 128)],
            sws[b],
        )

    fire(0, 0)
    fire(1, 1)

    def do_chunk(c, b):
        vb = vbufs[b]
        wb = wbufs[b]
        pltpu.make_async_copy(vtab_hbm.at[idx_l.at[c]], vb, svs[b]).wait()
        pltpu.make_async_copy(
            wtab_hbm.at[idxw_l.at[2 * c]], wb.at[pl.ds(0, 128)], sws[b]
        ).wait()
        pltpu.make_async_copy(
            wtab_hbm.at[idxw_l.at[2 * c + 1]], wb.at[pl.ds(128, 128)], sws[b]
        ).wait()
        for e in range(_NBE):
            r0 = e * _FB
            le = c * _NBE + e
            accs = tuple(
                lat_l[le, pl.ds(16 * t, 16)] for t in range(_NVREG)
            )
            accs = lax.fori_loop(
                0, _FB, lambda i, accs: _row_add(vb, r0 + i, accs), accs
            )

            g = accs[0]
            q = accs[0] * accs[0]
            for t in range(1, _NVREG):
                g = g + accs[t]
                q = q + accs[t] * accs[t]
            wa = wb[pl.ds(e * _WPAD, 16)]
            wbv = wb[pl.ds(e * _WPAD + 16, 16)]
            t_e = 0.5 * (g * g) - 0.5 * q + wa + jnp.where(lane_w, wbv, 0.0)
            eg = b * _NBE + e
            tb[pl.ds(eg * 16, 16)] = t_e
            gb[pl.ds(eg * 16, 16)] = g

        @pl.when(c + 2 < _NCHUNK)
        def _():
            fire(c + 2, b)

    def it(i, carry):
        do_chunk(2 * i, 0)
        do_chunk(2 * i + 1, 1)
        tcols = [plsc.load_gather(tb, [col_idx + l]) for l in range(16)]
        out_vec = tcols[0]
        for l in range(1, 16):
            out_vec = out_vec + tcols[l]
        gcols = [plsc.load_gather(gb, [col_idx + l]) for l in range(16)]
        for k in range(8):
            out_vec = out_vec + gcols[k] * gcols[k + 8]
        out_l[pl.ds(i * 16, 16)] = out_vec
        return carry

    lax.fori_loop(0, _NCHUNK // 2, it, 0)
    pltpu.sync_copy(out_l, out_hbm.at[pl.ds(base, _BPW)])


def _tr_body(x_ref, o_ref):
    xb = x_ref[...].astype(jnp.bfloat16)
    parts = []
    for u in range(_NPAIR):
        a = xb[32 * u:32 * u + 16, :]
        au = lax.bitcast_convert_type(a, jnp.uint16).astype(jnp.uint32)
        if 2 * u + 1 < _NVREG:
            b = xb[32 * u + 16:32 * u + 32, :]
            bu = lax.bitcast_convert_type(b, jnp.uint16).astype(jnp.uint32)
            au = au | (bu << 16)
        parts.append(au)
    o_ref[...] = jnp.concatenate(parts, axis=0).T


_transpose0 = pl.pallas_call(
    _tr_body,
    grid=(_NBLK0,),
    in_specs=[pl.BlockSpec((_D, _TBN), lambda j: (0, j))],
    out_specs=pl.BlockSpec((_TBN, _DU), lambda j: (j, 0)),
    out_shape=jax.ShapeDtypeStruct((_N0, _DU), jnp.uint32),
)

_transpose1 = pl.pallas_call(
    _tr_body,
    grid=(_NBLK1,),
    in_specs=[pl.BlockSpec((_D, _TBN), lambda j: (0, j + _NBLK0 - 1))],
    out_specs=pl.BlockSpec((_TBN, _DU), lambda j: (j, 0)),
    out_shape=jax.ShapeDtypeStruct((_N1, _DU), jnp.uint32),
)


@jax.jit
def kernel(inputs, w0, w, v):
    mapped = jnp.asarray(inputs, jnp.int32) + jnp.asarray(_OFFS)[None, :]
    idx0 = mapped[:, :_FB].reshape(_B * _FB // _CIDX, _CIDX)
    idx1 = (mapped[:, _FB:] - _SPLIT).reshape(_B * _FB // _CIDX, _CIDX)
    idxw = jnp.concatenate(
        [mapped, jnp.zeros((_B, _WPAD - _FIELD), jnp.int32)], axis=1
    ).reshape(_B * _WPAD // 128, 128)
    vt = v.reshape(_NROW, _D).T
    x0 = _transpose0(vt)
    x1 = _transpose1(vt)
    lat = _ffm_band0(idx0, x0)
    out = _ffm_band1(idx1, idxw, x1, w.reshape(_NROW), lat)
    return out.reshape(_B, 1) + w0
